# Initial kernel scaffold; baseline (speedup 1.0000x reference)
import jax
import jax.numpy as jnp
from jax.experimental import pallas as pl


def kernel(camera_indices, embedding_weight):
    raise NotImplementedError("write your pallas kernel here")


# speedup vs baseline: 2.9738x; 2.9738x over previous
import functools

import jax
import jax.numpy as jnp
from jax import lax
from jax.experimental import pallas as pl
from jax.experimental.pallas import tpu as pltpu
from jax.experimental.pallas import tpu_sc as plsc

_B, _S = 16384, 50
_N = _B * _S
_D = 32

_CH = 1600


@functools.lru_cache(maxsize=1)
def _build_gather():
    info = plsc.get_sparse_core_info()
    nc, ns = info.num_cores, info.num_subcores
    nw = nc * ns
    bpw = _N // nw
    nch = bpw // _CH

    mesh = plsc.VectorSubcoreMesh(core_axis_name="c", subcore_axis_name="s")

    @functools.partial(
        pl.kernel,
        out_type=jax.ShapeDtypeStruct((_N, _D), jnp.float32),
        mesh=mesh,
        scratch_types=[
            pltpu.VMEM((_CH,), jnp.int32),
            pltpu.VMEM((_CH, _D), jnp.float32),
            pltpu.SemaphoreType.DMA,
        ],
        compiler_params=pltpu.CompilerParams(use_tc_tiling_on_sc=False),
    )
    def gather_kernel(idx_hbm, table_hbm, out_hbm, idx_v, rows_v, sem):
        wid = lax.axis_index("s") * nc + lax.axis_index("c")
        base = wid * bpw
        for ch in range(nch):
            off = base + ch * _CH
            pltpu.sync_copy(idx_hbm.at[pl.ds(off, _CH)], idx_v)
            pltpu.async_copy(table_hbm.at[idx_v], rows_v, sem).wait()
            pltpu.sync_copy(rows_v, out_hbm.at[pl.ds(off, _CH)])

    return gather_kernel


def kernel(camera_indices, embedding_weight):
    idx = camera_indices.reshape(-1).astype(jnp.int32)
    out = _build_gather()(idx, embedding_weight)
    return out.reshape(camera_indices.shape + (_D,))


# speedup vs baseline: 3.0157x; 1.0141x over previous
import functools

import jax
import jax.numpy as jnp
from jax import lax
from jax.experimental import pallas as pl
from jax.experimental.pallas import tpu as pltpu
from jax.experimental.pallas import tpu_sc as plsc

_B, _S = 16384, 50
_N = _B * _S
_D = 32

_CH = 1024
_NBUF = 3


@functools.lru_cache(maxsize=1)
def _build_gather():
    info = plsc.get_sparse_core_info()
    nc, ns = info.num_cores, info.num_subcores
    nw = nc * ns
    bpw = _N // nw
    nch = bpw // _CH

    mesh = plsc.VectorSubcoreMesh(core_axis_name="c", subcore_axis_name="s")

    @functools.partial(
        pl.kernel,
        out_type=jax.ShapeDtypeStruct((_N, _D), jnp.float32),
        mesh=mesh,
        scratch_types=[
            pltpu.VMEM((bpw,), jnp.int32),
            *[pltpu.VMEM((_CH, _D), jnp.float32) for _ in range(_NBUF)],
            *[pltpu.SemaphoreType.DMA for _ in range(2 * _NBUF)],
        ],
        compiler_params=pltpu.CompilerParams(use_tc_tiling_on_sc=False),
    )
    def gather_kernel(idx_hbm, table_hbm, out_hbm, idx_v, *scratch):
        bufs = scratch[:_NBUF]
        gsem = scratch[_NBUF:2 * _NBUF]
        psem = scratch[2 * _NBUF:]
        wid = lax.axis_index("s") * nc + lax.axis_index("c")
        base = wid * bpw
        pltpu.sync_copy(idx_hbm.at[pl.ds(base, bpw)], idx_v)
        gets = [None] * _NBUF
        puts = [None] * _NBUF
        for j in range(min(_NBUF, nch)):
            gets[j] = pltpu.async_copy(
                table_hbm.at[idx_v.at[pl.ds(j * _CH, _CH)]], bufs[j], gsem[j]
            )
        for ch in range(nch):
            b = ch % _NBUF
            gets[b].wait()
            puts[b] = pltpu.async_copy(
                bufs[b], out_hbm.at[pl.ds(base + ch * _CH, _CH)], psem[b]
            )
            nxt = ch + _NBUF
            if nxt < nch:
                puts[b].wait()
                gets[b] = pltpu.async_copy(
                    table_hbm.at[idx_v.at[pl.ds(nxt * _CH, _CH)]], bufs[b], gsem[b]
                )
        for ch in range(max(0, nch - _NBUF), nch):
            puts[ch % _NBUF].wait()

    return gather_kernel


def kernel(camera_indices, embedding_weight):
    idx = camera_indices.reshape(-1).astype(jnp.int32)
    out = _build_gather()(idx, embedding_weight)
    return out.reshape(camera_indices.shape + (_D,))


# speedup vs baseline: 10.1941x; 3.3804x over previous
import functools

import jax
import jax.numpy as jnp
from jax import lax
from jax.experimental import pallas as pl
from jax.experimental.pallas import tpu as pltpu
from jax.experimental.pallas import tpu_sc as plsc

_B, _S = 16384, 50
_V, _D = 100000, 32

_CHB = 4096


@functools.lru_cache(maxsize=1)
def _build_lookup():
    info = plsc.get_sparse_core_info()
    nc, ns = info.num_cores, info.num_subcores
    assert nc * ns == _D, "one subcore per embedding dim"
    nbc = _B // _CHB

    mesh = plsc.VectorSubcoreMesh(core_axis_name="c", subcore_axis_name="s")

    @functools.partial(
        pl.kernel,
        out_type=jax.ShapeDtypeStruct((_S, _D, _B), jnp.float32),
        mesh=mesh,
        scratch_types=[
            pltpu.VMEM((_V,), jnp.float32),
            pltpu.VMEM((_CHB,), jnp.int32),
            pltpu.VMEM((_CHB,), jnp.int32),
            pltpu.VMEM((_CHB,), jnp.float32),
            pltpu.VMEM((_CHB,), jnp.float32),
            pltpu.SemaphoreType.DMA,
            pltpu.SemaphoreType.DMA,
            pltpu.SemaphoreType.DMA,
            pltpu.SemaphoreType.DMA,
        ],
        compiler_params=pltpu.CompilerParams(needs_layout_passes=False),
    )
    def lookup_kernel(idx_hbm, tbl_hbm, out_hbm, tbl_v, i0, i1, o0, o1,
                      gi0, gi1, po0, po1):
        d = lax.axis_index("s") * nc + lax.axis_index("c")
        pltpu.sync_copy(tbl_hbm.at[d, :], tbl_v)

        ibufs, isems = (i0, i1), (gi0, gi1)
        obufs, osems = (o0, o1), (po0, po1)
        chunks = [(s, bc) for s in range(_S) for bc in range(nbc)]

        def gather_chunk(ibuf, obuf):
            def body(j, _):
                iv = ibuf[pl.ds(j * 16, 16)]
                obuf[pl.ds(j * 16, 16)] = plsc.load_gather(tbl_v, [iv])
                return 0

            lax.fori_loop(0, _CHB // 16, body, 0)

        s0, bc0 = chunks[0]
        iget = [None, None]
        oput = [None, None]
        iget[0] = pltpu.async_copy(
            idx_hbm.at[s0, pl.ds(bc0 * _CHB, _CHB)], ibufs[0], isems[0]
        )
        for k, (s, bc) in enumerate(chunks):
            p = k % 2
            if k + 1 < len(chunks):
                sn, bcn = chunks[k + 1]
                iget[1 - p] = pltpu.async_copy(
                    idx_hbm.at[sn, pl.ds(bcn * _CHB, _CHB)],
                    ibufs[1 - p], isems[1 - p],
                )
            iget[p].wait()
            if oput[p] is not None:
                oput[p].wait()
            gather_chunk(ibufs[p], obufs[p])
            oput[p] = pltpu.async_copy(
                obufs[p], out_hbm.at[s, d, pl.ds(bc * _CHB, _CHB)], osems[p]
            )
        for p in range(2):
            if oput[p] is not None:
                oput[p].wait()

    return lookup_kernel


def kernel(camera_indices, embedding_weight):
    idx_t = camera_indices.T
    tbl_t = embedding_weight.T
    out_t = _build_lookup()(idx_t, tbl_t)
    return jnp.transpose(out_t, (2, 0, 1))


# speedup vs baseline: 13.9791x; 1.3713x over previous
import functools

import jax
import jax.numpy as jnp
from jax import lax
from jax.experimental import pallas as pl
from jax.experimental.pallas import tpu as pltpu
from jax.experimental.pallas import tpu_sc as plsc

_B, _S = 16384, 50
_V, _D = 100000, 32

_CHB = 4096


@functools.lru_cache(maxsize=1)
def _build_lookup():
    info = plsc.get_sparse_core_info()
    nc, ns = info.num_cores, info.num_subcores
    assert nc * ns == _D, "one subcore per embedding dim"
    nbc = _B // _CHB

    mesh = plsc.VectorSubcoreMesh(core_axis_name="c", subcore_axis_name="s")

    @functools.partial(
        pl.kernel,
        out_type=jax.ShapeDtypeStruct((_S, _D, _B), jnp.float32),
        mesh=mesh,
        scratch_types=[
            pltpu.VMEM((_V,), jnp.float32),
            pltpu.VMEM((_CHB,), jnp.int32),
            pltpu.VMEM((_CHB,), jnp.int32),
            pltpu.VMEM((_CHB,), jnp.float32),
            pltpu.VMEM((_CHB,), jnp.float32),
            pltpu.SemaphoreType.DMA,
            pltpu.SemaphoreType.DMA,
            pltpu.SemaphoreType.DMA,
            pltpu.SemaphoreType.DMA,
        ],
        compiler_params=pltpu.CompilerParams(needs_layout_passes=False),
    )
    def lookup_kernel(idx_hbm, tbl_hbm, out_hbm, tbl_v, i0, i1, o0, o1,
                      gi0, gi1, po0, po1):
        d = lax.axis_index("s") * nc + lax.axis_index("c")
        pltpu.sync_copy(tbl_hbm.at[d, :], tbl_v)

        nch = _S * nbc

        def islice(c):
            return idx_hbm.at[c // nbc, pl.ds((c % nbc) * _CHB, _CHB)]

        def oslice(c):
            return out_hbm.at[c // nbc, d, pl.ds((c % nbc) * _CHB, _CHB)]

        def gather_chunk(ibuf, obuf):
            def body(j, _):
                b = j * 128
                for u in range(8):
                    iv = ibuf[pl.ds(b + u * 16, 16)]
                    obuf[pl.ds(b + u * 16, 16)] = plsc.load_gather(
                        tbl_v, [iv]
                    )
                return 0

            lax.fori_loop(0, _CHB // 128, body, 0, unroll=False)

        pltpu.async_copy(islice(0), i0, gi0)

        def step(k, _):
            c0 = 2 * k
            pltpu.async_copy(islice(c0 + 1), i1, gi1)
            pltpu.make_async_copy(islice(c0), i0, gi0).wait()

            @pl.when(k > 0)
            def _():
                pltpu.make_async_copy(o0, oslice(c0), po0).wait()

            gather_chunk(i0, o0)
            pltpu.async_copy(o0, oslice(c0), po0)
            @pl.when(k < nch // 2 - 1)
            def _():
                pltpu.async_copy(islice(c0 + 2), i0, gi0)

            pltpu.make_async_copy(islice(c0 + 1), i1, gi1).wait()

            @pl.when(k > 0)
            def _():
                pltpu.make_async_copy(o1, oslice(c0 + 1), po1).wait()

            gather_chunk(i1, o1)
            pltpu.async_copy(o1, oslice(c0 + 1), po1)
            return 0

        lax.fori_loop(0, nch // 2, step, 0, unroll=False)
        pltpu.make_async_copy(o0, oslice(nch - 2), po0).wait()
        pltpu.make_async_copy(o1, oslice(nch - 1), po1).wait()

    return lookup_kernel


def kernel(camera_indices, embedding_weight):
    idx_t = camera_indices.T
    tbl_t = embedding_weight.T
    out_t = _build_lookup()(idx_t, tbl_t)
    return jnp.transpose(out_t, (2, 0, 1))


# speedup vs baseline: 19.3677x; 1.3855x over previous
import functools

import jax
import jax.numpy as jnp
from jax import lax
from jax.experimental import pallas as pl
from jax.experimental.pallas import tpu as pltpu
from jax.experimental.pallas import tpu_sc as plsc

_B, _S = 16384, 50
_V, _D = 100000, 32

_CHB = 4096


@functools.lru_cache(maxsize=1)
def _build_lookup():
    info = plsc.get_sparse_core_info()
    nc, ns = info.num_cores, info.num_subcores
    assert nc * ns == _D, "one subcore per embedding dim"
    nbc = _B // _CHB

    mesh = plsc.VectorSubcoreMesh(core_axis_name="c", subcore_axis_name="s")

    @functools.partial(
        pl.kernel,
        out_type=jax.ShapeDtypeStruct((_S, _D, _B), jnp.float32),
        mesh=mesh,
        scratch_types=[
            pltpu.VMEM((_V,), jnp.float32),
            pltpu.VMEM((_CHB,), jnp.int32),
            pltpu.VMEM((_CHB,), jnp.int32),
            pltpu.VMEM((_CHB,), jnp.float32),
            pltpu.VMEM((_CHB,), jnp.float32),
            pltpu.SemaphoreType.DMA,
            pltpu.SemaphoreType.DMA,
            pltpu.SemaphoreType.DMA,
            pltpu.SemaphoreType.DMA,
        ],
        compiler_params=pltpu.CompilerParams(needs_layout_passes=False),
    )
    def lookup_kernel(idx_hbm, tbl_hbm, out_hbm, tbl_v, i0, i1, o0, o1,
                      gi0, gi1, po0, po1):
        d = lax.axis_index("s") * nc + lax.axis_index("c")
        pltpu.sync_copy(tbl_hbm.at[d, :], tbl_v)

        nch = _S * nbc

        def islice(c):
            return idx_hbm.at[c // nbc, pl.ds((c % nbc) * _CHB, _CHB)]

        def oslice(c):
            return out_hbm.at[c // nbc, d, pl.ds((c % nbc) * _CHB, _CHB)]

        def gather_chunk(ibuf, obuf):
            def body(j, _):
                b = j * 256
                ivs = [ibuf[pl.ds(b + u * 16, 16)] for u in range(16)]
                vals = [plsc.load_gather(tbl_v, [iv]) for iv in ivs]
                for u in range(16):
                    obuf[pl.ds(b + u * 16, 16)] = vals[u]
                return 0

            lax.fori_loop(0, _CHB // 256, body, 0, unroll=False)

        pltpu.async_copy(islice(0), i0, gi0)

        def step(k, _):
            c0 = 2 * k
            pltpu.async_copy(islice(c0 + 1), i1, gi1)
            pltpu.make_async_copy(islice(c0), i0, gi0).wait()

            @pl.when(k > 0)
            def _():
                pltpu.make_async_copy(o0, oslice(c0), po0).wait()

            gather_chunk(i0, o0)
            pltpu.async_copy(o0, oslice(c0), po0)
            @pl.when(k < nch // 2 - 1)
            def _():
                pltpu.async_copy(islice(c0 + 2), i0, gi0)

            pltpu.make_async_copy(islice(c0 + 1), i1, gi1).wait()

            @pl.when(k > 0)
            def _():
                pltpu.make_async_copy(o1, oslice(c0 + 1), po1).wait()

            gather_chunk(i1, o1)
            pltpu.async_copy(o1, oslice(c0 + 1), po1)
            return 0

        lax.fori_loop(0, nch // 2, step, 0, unroll=False)
        pltpu.make_async_copy(o0, oslice(nch - 2), po0).wait()
        pltpu.make_async_copy(o1, oslice(nch - 1), po1).wait()

    return lookup_kernel


def kernel(camera_indices, embedding_weight):
    idx_t = camera_indices.T
    tbl_t = embedding_weight.T
    out_t = _build_lookup()(idx_t, tbl_t)
    return jnp.transpose(out_t, (2, 0, 1))
